# baseline (device time: 234393 ns/iter reference)
import jax
import jax.numpy as jnp
from jax import lax
from jax.experimental import pallas as pl
from jax.experimental.pallas import tpu as pltpu

N_DEV = 8
N_EXPERTS = 32
EPG = N_EXPERTS // N_DEV
CAPACITY = 409

_OFF = (-1, 1, 4, -2, 2, 3, -3)


def _ring_neighbors(my):
    left = lax.rem(my + N_DEV - 1, N_DEV)
    right = lax.rem(my + 1, N_DEV)
    return left, right


def _counts_broadcast(counts_pad):

    def body(c_ref, out_ref, send_sems, recv_sems):
        my = lax.axis_index("i")

        barrier_sem = pltpu.get_barrier_semaphore()
        for k in range(1, N_DEV):
            tgt = lax.rem(my + k, N_DEV)
            pl.semaphore_signal(
                barrier_sem, inc=1,
                device_id=(tgt,), device_id_type=pl.DeviceIdType.MESH,
            )
        pl.semaphore_wait(barrier_sem, N_DEV - 1)

        out_ref[pl.ds(my, 1), :] = c_ref[:, :]

        sends = []
        for k in range(1, N_DEV):
            tgt = lax.rem(my + k, N_DEV)
            rdma = pltpu.make_async_remote_copy(
                src_ref=c_ref,
                dst_ref=out_ref.at[pl.ds(my, 1), :],
                send_sem=send_sems.at[k - 1],
                recv_sem=recv_sems.at[k - 1],
                device_id=(tgt,),
                device_id_type=pl.DeviceIdType.MESH,
            )
            rdma.start()
            sends.append(rdma)
        for k in range(1, N_DEV):
            src = lax.rem(my + N_DEV - k, N_DEV)
            recv = pltpu.make_async_remote_copy(
                src_ref=c_ref,
                dst_ref=out_ref.at[pl.ds(src, 1), :],
                send_sem=send_sems.at[k - 1],
                recv_sem=recv_sems.at[k - 1],
                device_id=(src,),
                device_id_type=pl.DeviceIdType.MESH,
            )
            recv.wait_recv()
        for rdma in sends:
            rdma.wait_send()

    return pl.pallas_call(
        body,
        out_shape=jax.ShapeDtypeStruct((N_DEV, 128), jnp.float32),
        in_specs=[pl.BlockSpec(memory_space=pltpu.VMEM)],
        out_specs=pl.BlockSpec(memory_space=pltpu.VMEM),
        scratch_shapes=[
            pltpu.SemaphoreType.DMA((N_DEV - 1,)),
            pltpu.SemaphoreType.DMA((N_DEV - 1,)),
        ],
        compiler_params=pltpu.CompilerParams(collective_id=0),
    )(counts_pad)


def _moe_ringchord(x, sel, w_shard):
    n_tok, d_model = x.shape
    d_ff = w_shard.shape[-1]

    def body(x_ref, s_ref, w_ref, out_ref, comm, send_sems, recv_sems):
        my = lax.axis_index("i")
        left, right = _ring_neighbors(my)
        anti = lax.rem(my + 4, N_DEV)

        barrier_sem = pltpu.get_barrier_semaphore()
        for nbr in (left, right, anti):
            pl.semaphore_signal(
                barrier_sem, inc=1,
                device_id=(nbr,), device_id_type=pl.DeviceIdType.MESH,
            )
        pl.semaphore_wait(barrier_sem, 3)

        def send(src_ref, dst_slot, tgt):
            rdma = pltpu.make_async_remote_copy(
                src_ref=src_ref,
                dst_ref=comm.at[dst_slot],
                send_sem=send_sems.at[dst_slot],
                recv_sem=recv_sems.at[dst_slot],
                device_id=(tgt,),
                device_id_type=pl.DeviceIdType.MESH,
            )
            rdma.start()
            return rdma

        def wait_recv_slot(k):
            pltpu.make_async_remote_copy(
                src_ref=comm.at[k],
                dst_ref=comm.at[k],
                send_sem=send_sems.at[k],
                recv_sem=recv_sems.at[k],
                device_id=(left,),
                device_id_type=pl.DeviceIdType.MESH,
            ).wait_recv()

        def compute(off, load_j):
            g = lax.rem(my + off + N_DEV, N_DEV)
            m = s_ref[g]
            for j in range(EPG):
                out_ref[...] += jnp.dot(
                    x_ref[...] * m[:, j : j + 1],
                    load_j(j),
                    preferred_element_type=jnp.float32,
                )

        sends = [
            send(w_ref, 0, right),
            send(w_ref, 1, left),
            send(w_ref, 2, anti),
        ]
        out_ref[...] = jnp.zeros((n_tok, d_ff), jnp.float32)
        compute(0, lambda j: w_ref[j])

        wait_recv_slot(0)
        sends.append(send(comm.at[0], 3, right))
        wait_recv_slot(1)
        sends.append(send(comm.at[1], 4, left))
        wait_recv_slot(2)
        sends.append(send(comm.at[2], 5, right))
        sends.append(send(comm.at[2], 6, left))

        for k in (0, 1, 2):
            compute(_OFF[k], lambda j, _k=k: comm[_k, j])
        for k in (3, 4, 5, 6):
            wait_recv_slot(k)
            compute(_OFF[k], lambda j, _k=k: comm[_k, j])

        for rdma in sends:
            rdma.wait_send()

    return pl.pallas_call(
        body,
        out_shape=jax.ShapeDtypeStruct((n_tok, d_ff), jnp.float32),
        in_specs=[
            pl.BlockSpec(memory_space=pltpu.VMEM),
            pl.BlockSpec(memory_space=pltpu.VMEM),
            pl.BlockSpec(memory_space=pltpu.VMEM),
        ],
        out_specs=pl.BlockSpec(memory_space=pltpu.VMEM),
        scratch_shapes=[
            pltpu.VMEM((7, EPG, d_model, d_ff), jnp.bfloat16),
            pltpu.SemaphoreType.DMA((7,)),
            pltpu.SemaphoreType.DMA((7,)),
        ],
        compiler_params=pltpu.CompilerParams(collective_id=1),
    )(x, sel, w_shard)


def kernel(x, router_W, route_idx, expert_W):
    del router_W
    n_tok = x.shape[0]

    e = route_idx[:, 0]
    oh = (e[:, None] == jnp.arange(N_EXPERTS, dtype=e.dtype)[None, :]).astype(
        jnp.float32
    )

    counts_pad = jnp.zeros((1, 128), jnp.float32).at[0, :N_EXPERTS].set(oh.sum(0))
    all_counts = _counts_broadcast(counts_pad)

    my = lax.axis_index("i")
    prev = (jnp.arange(N_DEV) < my).astype(jnp.float32)
    base = (prev @ all_counts)[:N_EXPERTS]
    ranks = jnp.cumsum(oh, axis=0) - oh
    keep = ((base[None, :] + ranks) < CAPACITY) & (oh > 0)
    sel = keep.astype(jnp.float32)

    sel_g = sel.reshape(n_tok, N_DEV, EPG).transpose(1, 0, 2)
    sel_g = jnp.pad(sel_g, ((0, 0), (0, 0), (0, 128 - EPG)))

    return _moe_ringchord(
        x.astype(jnp.bfloat16),
        sel_g.astype(jnp.bfloat16),
        expert_W.astype(jnp.bfloat16),
    )


# device time: 201050 ns/iter; 1.1658x vs baseline; 1.1658x over previous
import jax
import jax.numpy as jnp
from jax import lax
from jax.experimental import pallas as pl
from jax.experimental.pallas import tpu as pltpu

N_DEV = 8
N_EXPERTS = 32
EPG = N_EXPERTS // N_DEV
CAPACITY = 409

COLS = ((0, 384), (384, 768), (768, 1024))
MASKS = ((1, 3, 4), (3, 4, 1), (4, 1, 3))


def _slot_origins(masks):
    m1, m2, m3 = masks
    return (0, m1, m2, m2 ^ m1, m3, m3 ^ m1, m3 ^ m2, m3 ^ m2 ^ m1)


def _counts_broadcast(counts_pad):

    def body(c_ref, out_ref, send_sems, recv_sems):
        my = lax.axis_index("i")

        barrier_sem = pltpu.get_barrier_semaphore()
        for k in range(1, N_DEV):
            tgt = lax.rem(my + k, N_DEV)
            pl.semaphore_signal(
                barrier_sem, inc=1,
                device_id=(tgt,), device_id_type=pl.DeviceIdType.MESH,
            )
        pl.semaphore_wait(barrier_sem, N_DEV - 1)

        out_ref[pl.ds(my, 1), :] = c_ref[:, :]

        sends = []
        for k in range(1, N_DEV):
            tgt = lax.rem(my + k, N_DEV)
            rdma = pltpu.make_async_remote_copy(
                src_ref=c_ref,
                dst_ref=out_ref.at[pl.ds(my, 1), :],
                send_sem=send_sems.at[k - 1],
                recv_sem=recv_sems.at[k - 1],
                device_id=(tgt,),
                device_id_type=pl.DeviceIdType.MESH,
            )
            rdma.start()
            sends.append(rdma)
        for k in range(1, N_DEV):
            src = lax.rem(my + N_DEV - k, N_DEV)
            recv = pltpu.make_async_remote_copy(
                src_ref=c_ref,
                dst_ref=out_ref.at[pl.ds(src, 1), :],
                send_sem=send_sems.at[k - 1],
                recv_sem=recv_sems.at[k - 1],
                device_id=(src,),
                device_id_type=pl.DeviceIdType.MESH,
            )
            recv.wait_recv()
        for rdma in sends:
            rdma.wait_send()

    return pl.pallas_call(
        body,
        out_shape=jax.ShapeDtypeStruct((N_DEV, 128), jnp.float32),
        in_specs=[pl.BlockSpec(memory_space=pltpu.VMEM)],
        out_specs=pl.BlockSpec(memory_space=pltpu.VMEM),
        scratch_shapes=[
            pltpu.SemaphoreType.DMA((N_DEV - 1,)),
            pltpu.SemaphoreType.DMA((N_DEV - 1,)),
        ],
        compiler_params=pltpu.CompilerParams(collective_id=0),
    )(counts_pad)


def _moe_cube_allgather(x, sel, w_shard):
    n_tok, d_model = x.shape
    d_ff = w_shard.shape[-1]

    def body(x_ref, s_ref, w_ref, out_ref, b0, b1, b2, send_sems, recv_sems):
        bufs = (b0, b1, b2)
        my = lax.axis_index("i")

        barrier_sem = pltpu.get_barrier_semaphore()
        for mask in (1, 3, 4):
            pl.semaphore_signal(
                barrier_sem, inc=1,
                device_id=(lax.bitwise_xor(my, mask),),
                device_id_type=pl.DeviceIdType.MESH,
            )
        pl.semaphore_wait(barrier_sem, 3)

        def descriptor(t, phase):
            n = 1 << phase
            return pltpu.make_async_remote_copy(
                src_ref=bufs[t].at[pl.ds(0, n)],
                dst_ref=bufs[t].at[pl.ds(n, n)],
                send_sem=send_sems.at[3 * t + phase],
                recv_sem=recv_sems.at[3 * t + phase],
                device_id=(lax.bitwise_xor(my, MASKS[t][phase]),),
                device_id_type=pl.DeviceIdType.MESH,
            )

        def compute_piece(t, k):
            lo, hi = COLS[t]
            m1, m2, m3 = MASKS[t]
            c = lax.bitwise_xor(
                lax.bitwise_xor(m1 * (k & 1), m2 * ((k >> 1) & 1)),
                m3 * ((k >> 2) & 1),
            )
            g = lax.bitwise_xor(my, c)
            m = s_ref[g]
            for j in range(EPG):
                out_ref[:, lo:hi] += jnp.dot(
                    x_ref[...] * m[:, j : j + 1],
                    bufs[t][k, j],
                    preferred_element_type=jnp.float32,
                ).astype(jnp.bfloat16)

        for t, (lo, hi) in enumerate(COLS):
            bufs[t][0] = w_ref[:, :, lo:hi]
        sends = [descriptor(t, 0) for t in range(3)]
        for rdma in sends:
            rdma.start()

        out_ref[...] = jnp.zeros((n_tok, d_ff), jnp.bfloat16)
        m_own = s_ref[my]
        for j in range(EPG):
            out_ref[...] += jnp.dot(
                x_ref[...] * m_own[:, j : j + 1],
                w_ref[j],
                preferred_element_type=jnp.float32,
            ).astype(jnp.bfloat16)

        order = (2, 0, 1)
        for phase in (1, 2):
            for t in order:
                descriptor(t, phase - 1).wait_recv()
                nxt = descriptor(t, phase)
                nxt.start()
                sends.append(nxt)
            for t in order:
                lax.fori_loop(
                    1 << (phase - 1),
                    1 << phase,
                    lambda k, _, _t=t: (compute_piece(_t, k), 0)[1],
                    0,
                )
        for t in order:
            descriptor(t, 2).wait_recv()
            lax.fori_loop(
                4, 8, lambda k, _, _t=t: (compute_piece(_t, k), 0)[1], 0
            )

        for rdma in sends:
            rdma.wait_send()

    piece_shapes = [
        pltpu.VMEM((N_DEV, EPG, d_model, hi - lo), jnp.bfloat16)
        for lo, hi in COLS
    ]
    return pl.pallas_call(
        body,
        out_shape=jax.ShapeDtypeStruct((n_tok, d_ff), jnp.bfloat16),
        in_specs=[
            pl.BlockSpec(memory_space=pltpu.VMEM),
            pl.BlockSpec(memory_space=pltpu.VMEM),
            pl.BlockSpec(memory_space=pltpu.VMEM),
        ],
        out_specs=pl.BlockSpec(memory_space=pltpu.VMEM),
        scratch_shapes=piece_shapes + [
            pltpu.SemaphoreType.DMA((9,)),
            pltpu.SemaphoreType.DMA((9,)),
        ],
        compiler_params=pltpu.CompilerParams(
            collective_id=1,
            vmem_limit_bytes=64 * 1024 * 1024,
        ),
    )(x, sel, w_shard)


def kernel(x, router_W, route_idx, expert_W):
    del router_W
    n_tok = x.shape[0]

    e = route_idx[:, 0]
    oh = (e[:, None] == jnp.arange(N_EXPERTS, dtype=e.dtype)[None, :]).astype(
        jnp.float32
    )

    counts_pad = jnp.zeros((1, 128), jnp.float32).at[0, :N_EXPERTS].set(oh.sum(0))
    all_counts = _counts_broadcast(counts_pad)

    my = lax.axis_index("i")
    prev = (jnp.arange(N_DEV) < my).astype(jnp.float32)
    base = (prev @ all_counts)[:N_EXPERTS]
    ranks = jnp.cumsum(oh, axis=0) - oh
    keep = ((base[None, :] + ranks) < CAPACITY) & (oh > 0)
    sel = keep.astype(jnp.float32)

    sel_g = sel.reshape(n_tok, N_DEV, EPG).transpose(1, 0, 2)
    sel_g = jnp.pad(sel_g, ((0, 0), (0, 0), (0, 128 - EPG)))

    out = _moe_cube_allgather(
        x.astype(jnp.bfloat16),
        sel_g.astype(jnp.bfloat16),
        expert_W.astype(jnp.bfloat16),
    )
    return out.astype(jnp.float32)
